# Initial kernel scaffold; baseline (speedup 1.0000x reference)
import jax
import jax.numpy as jnp
from jax.experimental import pallas as pl


def kernel(x, edge_index, W_self1, W_neigh1, b1, W_self2, W_neigh2, b2):
    raise NotImplementedError("write your pallas kernel here")


# speedup vs baseline: 5.8466x; 5.8466x over previous
import functools

import jax
import jax.numpy as jnp
from jax import lax
from jax.experimental import pallas as pl
from jax.experimental.pallas import tpu as pltpu
from jax.experimental.pallas import tpu_sc as plsc

N = 10000
E = 320000
D = 128
H = 128
C = 40
CP = 48

NC = 2
NS = 16
L = 16
NW = NC * NS
EPW = E // NW
CHUNK = 80
NCHUNK = EPW // CHUNK
RPT = N // NS

_mesh = plsc.VectorSubcoreMesh(core_axis_name="c", subcore_axis_name="s")


def _seg_sum_body(width, with_deg, x_hbm, src_hbm, dst_hbm, *rest):
    if with_deg:
        out_hbm, deg_hbm, src_v, dst_v, rows_v, deg_v, acc_sh, sem = rest
    else:
        out_hbm, src_v, dst_v, rows_v, acc_sh, sem = rest
        deg_hbm = deg_v = None
    cid = lax.axis_index("c")
    sid = lax.axis_index("s")
    wid = sid * NC + cid
    base = wid * EPW

    zeros = jnp.zeros((L,), jnp.float32)
    ones = jnp.ones((L,), jnp.float32)

    def zero_rows(i, _):
        r = i // (width // L)
        c = i % (width // L)
        rows_v[r, pl.ds(c * L, L)] = zeros
        return 0

    lax.fori_loop(0, CHUNK * (width // L), zero_rows, 0)
    for j in range(RPT // CHUNK):
        pltpu.sync_copy(rows_v, acc_sh.at[pl.ds(sid * RPT + j * CHUNK, CHUNK)])
    rem = RPT % CHUNK
    if rem:
        pltpu.sync_copy(rows_v.at[pl.ds(0, rem)],
                        acc_sh.at[pl.ds(sid * RPT + (RPT // CHUNK) * CHUNK, rem)])

    if with_deg:
        def zero_deg(i, _):
            deg_v[pl.ds(i * L, L)] = zeros
            return 0

        lax.fori_loop(0, N // L, zero_deg, 0)

    plsc.subcore_barrier()

    def chunk_body(g, _):
        off = base + g * CHUNK
        pltpu.sync_copy(src_hbm.at[pl.ds(off, CHUNK)], src_v)
        pltpu.sync_copy(dst_hbm.at[pl.ds(off, CHUNK)], dst_v)
        gat = pltpu.async_copy(x_hbm.at[src_v], rows_v, sem)
        if with_deg:
            for k in range(CHUNK // L):
                dv = dst_v[pl.ds(k * L, L)]
                plsc.addupdate_scatter(deg_v, [dv], ones)
        gat.wait()
        pltpu.sync_copy(rows_v, acc_sh.at[dst_v], add=True)
        return 0

    lax.fori_loop(0, NCHUNK, chunk_body, 0)

    if with_deg:
        pltpu.sync_copy(deg_v, deg_hbm.at[cid, sid])

    plsc.subcore_barrier()
    pltpu.sync_copy(acc_sh.at[pl.ds(sid * RPT, RPT)], out_hbm.at[cid, sid])


def _seg_sum_call(x, src, dst, width, with_deg):
    out_type = [jax.ShapeDtypeStruct((NC, NS, RPT, width), jnp.float32)]
    scratch = [
        pltpu.VMEM((CHUNK,), jnp.int32),
        pltpu.VMEM((CHUNK,), jnp.int32),
        pltpu.VMEM((CHUNK, width), jnp.float32),
    ]
    if with_deg:
        out_type.append(jax.ShapeDtypeStruct((NC, NS, N), jnp.float32))
        scratch.append(pltpu.VMEM((N,), jnp.float32))
    scratch.append(pltpu.VMEM_SHARED((N, width), jnp.float32))
    scratch.append(pltpu.SemaphoreType.DMA)
    fn = pl.kernel(
        functools.partial(_seg_sum_body, width, with_deg),
        out_type=out_type,
        mesh=_mesh,
        scratch_types=scratch,
        compiler_params=pltpu.CompilerParams(needs_layout_passes=False,
                                             use_tc_tiling_on_sc=False),
    )
    return fn(x, src, dst)


def _layer1_tc(x, aggx, degp, W_self1, W_neigh1, b1, Wn2p, Ws2p):
    R = 1000

    def body(x_b, aggx_b, degp_b, ws1, wn1, b1_b, wn2, ws2, y2_b, hs2_b):
        deg = jnp.sum(degp_b[...], axis=1)
        inv = 1.0 / jnp.clip(deg, 1.0, None)
        mean = (aggx_b[0] + aggx_b[1]) * inv[:, None]
        h1 = jnp.dot(x_b[...], ws1[...], preferred_element_type=jnp.float32,
                     precision=lax.Precision.HIGHEST)
        h1 += jnp.dot(mean, wn1[...], preferred_element_type=jnp.float32,
                      precision=lax.Precision.HIGHEST)
        h1 = jnp.maximum(h1 + b1_b[...], 0.0)
        y2_b[...] = jnp.dot(h1, wn2[...], preferred_element_type=jnp.float32,
                            precision=lax.Precision.HIGHEST)
        hs2_b[...] = jnp.dot(h1, ws2[...], preferred_element_type=jnp.float32,
                             precision=lax.Precision.HIGHEST)

    grid = (N // R,)
    return pl.pallas_call(
        body,
        grid=grid,
        in_specs=[
            pl.BlockSpec((R, D), lambda i: (i, 0)),
            pl.BlockSpec((NC, R, D), lambda i: (0, i, 0)),
            pl.BlockSpec((R, NW), lambda i: (i, 0)),
            pl.BlockSpec((D, H), lambda i: (0, 0)),
            pl.BlockSpec((D, H), lambda i: (0, 0)),
            pl.BlockSpec((1, H), lambda i: (0, 0)),
            pl.BlockSpec((H, CP), lambda i: (0, 0)),
            pl.BlockSpec((H, CP), lambda i: (0, 0)),
        ],
        out_specs=[
            pl.BlockSpec((R, CP), lambda i: (i, 0)),
            pl.BlockSpec((R, CP), lambda i: (i, 0)),
        ],
        out_shape=[
            jax.ShapeDtypeStruct((N, CP), jnp.float32),
            jax.ShapeDtypeStruct((N, CP), jnp.float32),
        ],
    )(x, aggx, degp, W_self1, W_neigh1, b1, Wn2p, Ws2p)


def _layer2_tc(hs2, agg2, degp, b2p):
    R = 2000

    def body(hs2_b, agg2_b, degp_b, b2_b, out_b):
        deg = jnp.sum(degp_b[...], axis=1)
        inv = 1.0 / jnp.clip(deg, 1.0, None)
        out_b[...] = hs2_b[...] + (agg2_b[0] + agg2_b[1]) * inv[:, None] + b2_b[...]

    grid = (N // R,)
    return pl.pallas_call(
        body,
        grid=grid,
        in_specs=[
            pl.BlockSpec((R, CP), lambda i: (i, 0)),
            pl.BlockSpec((NC, R, CP), lambda i: (0, i, 0)),
            pl.BlockSpec((R, NW), lambda i: (i, 0)),
            pl.BlockSpec((1, CP), lambda i: (0, 0)),
        ],
        out_specs=pl.BlockSpec((R, CP), lambda i: (i, 0)),
        out_shape=jax.ShapeDtypeStruct((N, CP), jnp.float32),
    )(hs2, agg2, degp, b2p)


def kernel(x, edge_index, W_self1, W_neigh1, b1, W_self2, W_neigh2, b2):
    src = edge_index[0]
    dst = edge_index[1]
    Wn2p = jnp.pad(W_neigh2, ((0, 0), (0, CP - C)))
    Ws2p = jnp.pad(W_self2, ((0, 0), (0, CP - C)))
    b1r = b1.reshape(1, H)
    b2p = jnp.pad(b2, (0, CP - C)).reshape(1, CP)

    aggx, degp = _seg_sum_call(x, src, dst, D, True)
    aggx = aggx.reshape(NC, N, D)
    degp_t = degp.reshape(NW, N).T
    y2p, hs2 = _layer1_tc(x, aggx, degp_t, W_self1, W_neigh1, b1r, Wn2p, Ws2p)
    (agg2,) = _seg_sum_call(y2p, src, dst, CP, False)
    agg2 = agg2.reshape(NC, N, CP)
    out48 = _layer2_tc(hs2, agg2, degp_t, b2p)
    return out48[:, :C]


# speedup vs baseline: 14.2875x; 2.4437x over previous
import functools

import jax
import jax.numpy as jnp
from jax import lax
from jax.experimental import pallas as pl
from jax.experimental.pallas import tpu as pltpu
from jax.experimental.pallas import tpu_sc as plsc

N = 10000
E = 320000
D = 128
H = 128
C = 40
CP = 48

NC = 2
NS = 16
L = 16
NW = NC * NS
EPW = E // NW
RPT = N // NS

_mesh = plsc.VectorSubcoreMesh(core_axis_name="c", subcore_axis_name="s")


def _seg_sum_body(width, chunk, nb, with_deg, x_hbm, src_hbm, dst_hbm, *rest):
    nchunk = EPW // chunk
    if with_deg:
        out_hbm, deg_hbm = rest[:2]
        rest = rest[2:]
    else:
        out_hbm = rest[0]
        deg_hbm = None
        rest = rest[1:]
    src_v, dst_v = rest[:2]
    rows = list(rest[2:2 + nb])
    gsems = list(rest[2 + nb:2 + 2 * nb])
    rest = rest[2 + 2 * nb:]
    if with_deg:
        deg_v = rest[0]
        acc_sh = rest[1]
    else:
        deg_v = None
        acc_sh = rest[0]

    cid = lax.axis_index("c")
    sid = lax.axis_index("s")
    wid = sid * NC + cid

    zeros = jnp.zeros((L,), jnp.float32)
    ones = jnp.ones((L,), jnp.float32)

    pltpu.sync_copy(src_hbm.at[pl.ds(wid * nchunk, nchunk)], src_v)
    pltpu.sync_copy(dst_hbm.at[pl.ds(wid * nchunk, nchunk)], dst_v)

    def zero_rows(i, _):
        r = i // (width // L)
        c = i % (width // L)
        rows[0][r, pl.ds(c * L, L)] = zeros
        return 0

    lax.fori_loop(0, chunk * (width // L), zero_rows, 0)
    for j in range(RPT // chunk):
        pltpu.sync_copy(rows[0], acc_sh.at[pl.ds(sid * RPT + j * chunk, chunk)])
    rem = RPT % chunk
    if rem:
        pltpu.sync_copy(rows[0].at[pl.ds(0, rem)],
                        acc_sh.at[pl.ds(sid * RPT + (RPT // chunk) * chunk, rem)])

    if with_deg:
        def zero_deg(i, _):
            deg_v[pl.ds(i * L, L)] = zeros
            return 0

        lax.fori_loop(0, N // L, zero_deg, 0)

    plsc.subcore_barrier()

    for b in range(nb):
        pltpu.async_copy(x_hbm.at[src_v.at[b]], rows[b], gsems[b])

    def deg_update(g):
        if with_deg:
            for k in range(chunk // L):
                dv = dst_v[g, pl.ds(k * L, L)]
                plsc.addupdate_scatter(deg_v, [dv], ones)

    nouter = nchunk // nb

    def outer(o, _):
        for b in range(nb):
            g = o * nb + b
            deg_update(g)
            pltpu.make_async_copy(x_hbm.at[src_v.at[g]], rows[b],
                                  gsems[b]).wait()
            pltpu.sync_copy(rows[b], acc_sh.at[dst_v.at[g]], add=True)
            pltpu.async_copy(x_hbm.at[src_v.at[g + nb]], rows[b], gsems[b])
        return 0

    lax.fori_loop(0, nouter - 1, outer, 0)
    for b in range(nb):
        g = (nouter - 1) * nb + b
        deg_update(g)
        pltpu.make_async_copy(x_hbm.at[src_v.at[g]], rows[b], gsems[b]).wait()
        pltpu.sync_copy(rows[b], acc_sh.at[dst_v.at[g]], add=True)
    for t in range(nchunk - nouter * nb):
        g = nouter * nb + t
        deg_update(g)
        pltpu.async_copy(x_hbm.at[src_v.at[g]], rows[0], gsems[0]).wait()
        pltpu.sync_copy(rows[0], acc_sh.at[dst_v.at[g]], add=True)

    if with_deg:
        pltpu.sync_copy(deg_v, deg_hbm.at[cid, sid])

    plsc.subcore_barrier()
    pltpu.sync_copy(acc_sh.at[pl.ds(sid * RPT, RPT)], out_hbm.at[cid, sid])


def _seg_sum_call(x, src2d, dst2d, width, chunk, nb, with_deg):
    nchunk = EPW // chunk
    out_type = [jax.ShapeDtypeStruct((NC, NS, RPT, width), jnp.float32)]
    if with_deg:
        out_type.append(jax.ShapeDtypeStruct((NC, NS, N), jnp.float32))
    scratch = [
        pltpu.VMEM((nchunk, chunk), jnp.int32),
        pltpu.VMEM((nchunk, chunk), jnp.int32),
    ]
    scratch += [pltpu.VMEM((chunk, width), jnp.float32) for _ in range(nb)]
    scratch += [pltpu.SemaphoreType.DMA for _ in range(nb)]
    if with_deg:
        scratch.append(pltpu.VMEM((N,), jnp.float32))
    scratch.append(pltpu.VMEM_SHARED((N, width), jnp.float32))
    fn = pl.kernel(
        functools.partial(_seg_sum_body, width, chunk, nb, with_deg),
        out_type=out_type,
        mesh=_mesh,
        scratch_types=scratch,
        compiler_params=pltpu.CompilerParams(needs_layout_passes=False,
                                             use_tc_tiling_on_sc=False),
    )
    return fn(x, src2d, dst2d)


def _layer1_tc(x, aggx, degp, W_self1, W_neigh1, b1, Wn2p, Ws2p):
    R = 1000

    def body(x_b, aggx_b, degp_b, ws1, wn1, b1_b, wn2, ws2, y2_b, hs2_b, inv_b):
        deg = jnp.sum(degp_b[...], axis=1)
        inv = 1.0 / jnp.clip(deg, 1.0, None)
        mean = (aggx_b[0] + aggx_b[1]) * inv[:, None]
        h1 = jnp.dot(x_b[...], ws1[...], preferred_element_type=jnp.float32,
                     precision=lax.Precision.HIGHEST)
        h1 += jnp.dot(mean, wn1[...], preferred_element_type=jnp.float32,
                      precision=lax.Precision.HIGHEST)
        h1 = jnp.maximum(h1 + b1_b[...], 0.0)
        y2_b[...] = jnp.dot(h1, wn2[...], preferred_element_type=jnp.float32,
                            precision=lax.Precision.HIGHEST)
        hs2_b[...] = jnp.dot(h1, ws2[...], preferred_element_type=jnp.float32,
                             precision=lax.Precision.HIGHEST)
        inv_b[...] = jnp.broadcast_to(inv[:, None], (R, 8))

    grid = (N // R,)
    return pl.pallas_call(
        body,
        grid=grid,
        in_specs=[
            pl.BlockSpec((R, D), lambda i: (i, 0)),
            pl.BlockSpec((NC, R, D), lambda i: (0, i, 0)),
            pl.BlockSpec((R, NW), lambda i: (i, 0)),
            pl.BlockSpec((D, H), lambda i: (0, 0)),
            pl.BlockSpec((D, H), lambda i: (0, 0)),
            pl.BlockSpec((1, H), lambda i: (0, 0)),
            pl.BlockSpec((H, CP), lambda i: (0, 0)),
            pl.BlockSpec((H, CP), lambda i: (0, 0)),
        ],
        out_specs=[
            pl.BlockSpec((R, CP), lambda i: (i, 0)),
            pl.BlockSpec((R, CP), lambda i: (i, 0)),
            pl.BlockSpec((R, 8), lambda i: (i, 0)),
        ],
        out_shape=[
            jax.ShapeDtypeStruct((N, CP), jnp.float32),
            jax.ShapeDtypeStruct((N, CP), jnp.float32),
            jax.ShapeDtypeStruct((N, 8), jnp.float32),
        ],
    )(x, aggx, degp, W_self1, W_neigh1, b1, Wn2p, Ws2p)


def _layer2_tc(hs2, agg2, invd, b2p):
    R = 2000

    def body(hs2_b, agg2_b, inv_b, b2_b, out_b):
        inv = inv_b[:, 0]
        out_b[...] = hs2_b[...] + (agg2_b[0] + agg2_b[1]) * inv[:, None] + b2_b[...]

    grid = (N // R,)
    return pl.pallas_call(
        body,
        grid=grid,
        in_specs=[
            pl.BlockSpec((R, CP), lambda i: (i, 0)),
            pl.BlockSpec((NC, R, CP), lambda i: (0, i, 0)),
            pl.BlockSpec((R, 8), lambda i: (i, 0)),
            pl.BlockSpec((1, CP), lambda i: (0, 0)),
        ],
        out_specs=pl.BlockSpec((R, CP), lambda i: (i, 0)),
        out_shape=jax.ShapeDtypeStruct((N, CP), jnp.float32),
    )(hs2, agg2, invd, b2p)


def kernel(x, edge_index, W_self1, W_neigh1, b1, W_self2, W_neigh2, b2):
    src = edge_index[0]
    dst = edge_index[1]
    Wn2p = jnp.pad(W_neigh2, ((0, 0), (0, CP - C)))
    Ws2p = jnp.pad(W_self2, ((0, 0), (0, CP - C)))
    b1r = b1.reshape(1, H)
    b2p = jnp.pad(b2, (0, CP - C)).reshape(1, CP)

    c1, nb1 = 80, 2
    c2, nb2 = 125, 4
    aggx, degp = _seg_sum_call(x, src.reshape(E // c1, c1),
                               dst.reshape(E // c1, c1), D, c1, nb1, True)
    aggx = aggx.reshape(NC, N, D)
    degp_t = degp.reshape(NW, N).T
    y2p, hs2, invd = _layer1_tc(x, aggx, degp_t, W_self1, W_neigh1, b1r,
                                Wn2p, Ws2p)
    agg2, = _seg_sum_call(y2p, src.reshape(E // c2, c2),
                          dst.reshape(E // c2, c2), CP, c2, nb2, False)
    agg2 = agg2.reshape(NC, N, CP)
    out48 = _layer2_tc(hs2, agg2, invd, b2p)
    return out48[:, :C]


# speedup vs baseline: 15.6069x; 1.0923x over previous
import functools

import jax
import jax.numpy as jnp
from jax import lax
from jax.experimental import pallas as pl
from jax.experimental.pallas import tpu as pltpu
from jax.experimental.pallas import tpu_sc as plsc

N = 10000
E = 320000
D = 128
H = 128
C = 40
CP = 48

NC = 2
NS = 16
L = 16
NW = NC * NS
EPW = E // NW
RPT = N // NS

_mesh = plsc.VectorSubcoreMesh(core_axis_name="c", subcore_axis_name="s")


def _seg_sum_body(width, chunk, nb, with_deg, x_hbm, src_hbm, dst_hbm, *rest):
    nchunk = EPW // chunk
    if with_deg:
        out_hbm, deg_hbm = rest[:2]
        rest = rest[2:]
    else:
        out_hbm = rest[0]
        deg_hbm = None
        rest = rest[1:]
    src_v, dst_v = rest[:2]
    rows = list(rest[2:2 + nb])
    gsems = list(rest[2 + nb:2 + 2 * nb])
    rest = rest[2 + 2 * nb:]
    if with_deg:
        deg_v = rest[0]
        acc_sh = rest[1]
    else:
        deg_v = None
        acc_sh = rest[0]

    cid = lax.axis_index("c")
    sid = lax.axis_index("s")
    wid = sid * NC + cid

    zeros = jnp.zeros((L,), jnp.float32)
    ones = jnp.ones((L,), jnp.float32)

    pltpu.sync_copy(src_hbm.at[pl.ds(wid * nchunk, nchunk)], src_v)
    pltpu.sync_copy(dst_hbm.at[pl.ds(wid * nchunk, nchunk)], dst_v)

    def zero_rows(i, _):
        r = i // (width // L)
        c = i % (width // L)
        rows[0][r, pl.ds(c * L, L)] = zeros
        return 0

    lax.fori_loop(0, chunk * (width // L), zero_rows, 0)
    for j in range(RPT // chunk):
        pltpu.sync_copy(rows[0], acc_sh.at[pl.ds(sid * RPT + j * chunk, chunk)])
    rem = RPT % chunk
    if rem:
        pltpu.sync_copy(rows[0].at[pl.ds(0, rem)],
                        acc_sh.at[pl.ds(sid * RPT + (RPT // chunk) * chunk, rem)])

    if with_deg:
        def zero_deg(i, _):
            deg_v[pl.ds(i * L, L)] = zeros
            return 0

        lax.fori_loop(0, N // L, zero_deg, 0)

    plsc.subcore_barrier()

    for b in range(nb):
        pltpu.async_copy(x_hbm.at[src_v.at[b]], rows[b], gsems[b])

    def deg_update(g):
        if with_deg:
            for k in range(chunk // L):
                dv = dst_v[g, pl.ds(k * L, L)]
                plsc.addupdate_scatter(deg_v, [dv], ones)

    nouter = nchunk // nb

    def outer(o, _):
        for b in range(nb):
            g = o * nb + b
            deg_update(g)
            pltpu.make_async_copy(x_hbm.at[src_v.at[g]], rows[b],
                                  gsems[b]).wait()
            pltpu.sync_copy(rows[b], acc_sh.at[dst_v.at[g]], add=True)
            pltpu.async_copy(x_hbm.at[src_v.at[g + nb]], rows[b], gsems[b])
        return 0

    lax.fori_loop(0, nouter - 1, outer, 0)
    for b in range(nb):
        g = (nouter - 1) * nb + b
        deg_update(g)
        pltpu.make_async_copy(x_hbm.at[src_v.at[g]], rows[b], gsems[b]).wait()
        pltpu.sync_copy(rows[b], acc_sh.at[dst_v.at[g]], add=True)
    for t in range(nchunk - nouter * nb):
        g = nouter * nb + t
        deg_update(g)
        pltpu.async_copy(x_hbm.at[src_v.at[g]], rows[0], gsems[0]).wait()
        pltpu.sync_copy(rows[0], acc_sh.at[dst_v.at[g]], add=True)

    if with_deg:
        pltpu.sync_copy(deg_v, deg_hbm.at[cid, sid])

    plsc.subcore_barrier()
    pltpu.sync_copy(acc_sh.at[pl.ds(sid * RPT, RPT)], out_hbm.at[cid, sid])


def _seg_sum_call(x, src2d, dst2d, width, chunk, nb, with_deg):
    nchunk = EPW // chunk
    out_type = [jax.ShapeDtypeStruct((NC, NS, RPT, width), jnp.float32)]
    if with_deg:
        out_type.append(jax.ShapeDtypeStruct((NC, NS, N), jnp.float32))
    scratch = [
        pltpu.VMEM((nchunk, chunk), jnp.int32),
        pltpu.VMEM((nchunk, chunk), jnp.int32),
    ]
    scratch += [pltpu.VMEM((chunk, width), jnp.float32) for _ in range(nb)]
    scratch += [pltpu.SemaphoreType.DMA for _ in range(nb)]
    if with_deg:
        scratch.append(pltpu.VMEM((N,), jnp.float32))
    scratch.append(pltpu.VMEM_SHARED((N, width), jnp.float32))
    fn = pl.kernel(
        functools.partial(_seg_sum_body, width, chunk, nb, with_deg),
        out_type=out_type,
        mesh=_mesh,
        scratch_types=scratch,
        compiler_params=pltpu.CompilerParams(needs_layout_passes=False,
                                             use_tc_tiling_on_sc=False),
    )
    return fn(x, src2d, dst2d)


def _layer1_tc(x, aggx, degp, W_self1, W_neigh1, b1, Wn2p, Ws2p):
    R = 2000

    def body(x_b, aggx_b, degp_b, ws1, wn1, b1_b, wn2, ws2, y2_b, hs2_b, inv_b):
        deg = jnp.sum(degp_b[...], axis=1)
        inv = 1.0 / jnp.clip(deg, 1.0, None)
        mean = (aggx_b[0] + aggx_b[1]) * inv[:, None]
        h1 = jnp.dot(x_b[...], ws1[...], preferred_element_type=jnp.float32)
        h1 += jnp.dot(mean, wn1[...], preferred_element_type=jnp.float32)
        h1 = jnp.maximum(h1 + b1_b[...], 0.0)
        y2_b[...] = jnp.dot(h1, wn2[...], preferred_element_type=jnp.float32)
        hs2_b[...] = jnp.dot(h1, ws2[...], preferred_element_type=jnp.float32)
        inv_b[...] = jnp.broadcast_to(inv[:, None], (R, 8))

    grid = (N // R,)
    return pl.pallas_call(
        body,
        grid=grid,
        in_specs=[
            pl.BlockSpec((R, D), lambda i: (i, 0)),
            pl.BlockSpec((NC, R, D), lambda i: (0, i, 0)),
            pl.BlockSpec((R, NW), lambda i: (i, 0)),
            pl.BlockSpec((D, H), lambda i: (0, 0)),
            pl.BlockSpec((D, H), lambda i: (0, 0)),
            pl.BlockSpec((1, H), lambda i: (0, 0)),
            pl.BlockSpec((H, CP), lambda i: (0, 0)),
            pl.BlockSpec((H, CP), lambda i: (0, 0)),
        ],
        out_specs=[
            pl.BlockSpec((R, CP), lambda i: (i, 0)),
            pl.BlockSpec((R, CP), lambda i: (i, 0)),
            pl.BlockSpec((R, 8), lambda i: (i, 0)),
        ],
        out_shape=[
            jax.ShapeDtypeStruct((N, CP), jnp.float32),
            jax.ShapeDtypeStruct((N, CP), jnp.float32),
            jax.ShapeDtypeStruct((N, 8), jnp.float32),
        ],
    )(x, aggx, degp, W_self1, W_neigh1, b1, Wn2p, Ws2p)


def _layer2_tc(hs2, agg2, invd, b2p):
    R = 2000

    def body(hs2_b, agg2_b, inv_b, b2_b, out_b):
        inv = inv_b[:, 0]
        full = hs2_b[...] + (agg2_b[0] + agg2_b[1]) * inv[:, None] + b2_b[...]
        out_b[...] = full[:, :C]

    grid = (N // R,)
    return pl.pallas_call(
        body,
        grid=grid,
        in_specs=[
            pl.BlockSpec((R, CP), lambda i: (i, 0)),
            pl.BlockSpec((NC, R, CP), lambda i: (0, i, 0)),
            pl.BlockSpec((R, 8), lambda i: (i, 0)),
            pl.BlockSpec((1, CP), lambda i: (0, 0)),
        ],
        out_specs=pl.BlockSpec((R, C), lambda i: (i, 0)),
        out_shape=jax.ShapeDtypeStruct((N, C), jnp.float32),
    )(hs2, agg2, invd, b2p)


def kernel(x, edge_index, W_self1, W_neigh1, b1, W_self2, W_neigh2, b2):
    src = edge_index[0]
    dst = edge_index[1]
    Wn2p = jnp.pad(W_neigh2, ((0, 0), (0, CP - C)))
    Ws2p = jnp.pad(W_self2, ((0, 0), (0, CP - C)))
    b1r = b1.reshape(1, H)
    b2p = jnp.pad(b2, (0, CP - C)).reshape(1, CP)

    c1, nb1 = 80, 2
    c2, nb2 = 125, 4
    aggx, degp = _seg_sum_call(x, src.reshape(E // c1, c1),
                               dst.reshape(E // c1, c1), D, c1, nb1, True)
    aggx = aggx.reshape(NC, N, D)
    degp_t = degp.reshape(NW, N).T
    y2p, hs2, invd = _layer1_tc(x, aggx, degp_t, W_self1, W_neigh1, b1r,
                                Wn2p, Ws2p)
    agg2, = _seg_sum_call(y2p, src.reshape(E // c2, c2),
                          dst.reshape(E // c2, c2), CP, c2, nb2, False)
    agg2 = agg2.reshape(NC, N, CP)
    return _layer2_tc(hs2, agg2, invd, b2p)
